# Initial kernel scaffold; baseline (speedup 1.0000x reference)
import jax
import jax.numpy as jnp
from jax.experimental import pallas as pl


def kernel(r, Z, composition_probs, num_atoms, alphas, type_sigmas):
    raise NotImplementedError("write your pallas kernel here")


# speedup vs baseline: 2.3829x; 2.3829x over previous
import jax
import jax.numpy as jnp
from jax import lax
from jax.experimental import pallas as pl
from jax.experimental.pallas import tpu as pltpu

_MAXA = 100
_T = 1000


def _seg_stats_kernel(starts_ref, ends_ref, eps_ref, out_ref, acc_ref):
    i = pl.program_id(0)
    nb = pl.num_programs(0)
    blk = eps_ref.shape[0]

    @pl.when(i == 0)
    def _init():
        acc_ref[...] = jnp.zeros_like(acc_ref)

    rows = lax.broadcasted_iota(jnp.int32, (blk, 16), 0) + i * blk
    oh = ((rows >= starts_ref[...]) & (rows < ends_ref[...])).astype(jnp.float32)
    acc_ref[:, 0:3] += lax.dot_general(
        oh, eps_ref[...], (((0,), (0,)), ((), ())),
        precision=lax.Precision.HIGHEST,
        preferred_element_type=jnp.float32)

    @pl.when(i == nb - 1)
    def _fin():
        out_ref[...] = jnp.zeros_like(out_ref)
        out_ref[0:16, 0:3] = acc_ref[:, 0:3]


def _main_kernel(starts_ref, ends_ref, inv_na_ref, alpha_ref, sigma_ref,
                 sums_ref, r_ref, eps_ref, z_ref, comp_ref,
                 rt_ref, epsr_ref, as_ref, table_ref):
    i = pl.program_id(0)
    blk = r_ref.shape[0]

    @pl.when(i == 0)
    def _build():
        ssum = sums_ref[0:16, 0:3] + sums_ref[16:32, 0:3]
        mean = ssum * inv_na_ref[...]
        alpha = alpha_ref[...]
        table_ref[...] = jnp.concatenate(
            [jnp.sqrt(alpha), jnp.sqrt(1.0 - alpha), sigma_ref[...], mean,
             jnp.zeros((16, 2), jnp.float32)], axis=1)

    rows = lax.broadcasted_iota(jnp.int32, (blk, 16), 0) + i * blk
    oh = ((rows >= starts_ref[...]) & (rows < ends_ref[...])).astype(jnp.float32)
    vals = jnp.dot(oh, table_ref[...], precision=lax.Precision.HIGHEST,
                   preferred_element_type=jnp.float32)
    epsr = eps_ref[...] - vals[:, 3:6]
    rt_ref[...] = vals[:, 0:1] * r_ref[...] + vals[:, 1:2] * epsr
    epsr_ref[...] = epsr
    cio = lax.broadcasted_iota(jnp.int32, (blk, _MAXA), 1)
    as_ref[...] = ((cio == (z_ref[...] - 1)).astype(jnp.float32)
                   + comp_ref[...] * vals[:, 2:3])


def kernel(r, Z, composition_probs, num_atoms, alphas, type_sigmas):
    N = r.shape[0]
    B = num_atoms.shape[0]
    key = jax.random.key(1)
    kt, ks, ke = jax.random.split(key, 3)
    t = jax.random.randint(kt, (B,), 1, _T)
    s = jax.random.randint(ks, (B,), 1, _T)
    eps = jax.random.normal(ke, (N, 3), dtype=jnp.float32)

    ends = jnp.cumsum(num_atoms, dtype=jnp.int32)
    starts = ends - num_atoms
    starts_row = starts.reshape(1, B)
    ends_row = ends.reshape(1, B)
    inv_na_col = (1.0 / num_atoms.astype(jnp.float32)).reshape(B, 1)
    alpha_col = alphas[t].reshape(B, 1)
    sigma_col = type_sigmas[s].reshape(B, 1)

    BLK = 2048
    nb = N // BLK

    sums = pl.pallas_call(
        _seg_stats_kernel,
        grid=(nb,),
        in_specs=[
            pl.BlockSpec((1, B), lambda i: (0, 0)),
            pl.BlockSpec((1, B), lambda i: (0, 0)),
            pl.BlockSpec((BLK, 3), lambda i: (i, 0)),
        ],
        out_specs=pl.BlockSpec((2 * B, B), lambda i: (0, 0)),
        out_shape=jax.ShapeDtypeStruct((2 * B, B), jnp.float32),
        scratch_shapes=[pltpu.VMEM((B, 8), jnp.float32)],
    )(starts_row, ends_row, eps)

    rt, epsr, a_s = pl.pallas_call(
        _main_kernel,
        grid=(nb,),
        in_specs=[
            pl.BlockSpec((1, B), lambda i: (0, 0)),
            pl.BlockSpec((1, B), lambda i: (0, 0)),
            pl.BlockSpec((B, 1), lambda i: (0, 0)),
            pl.BlockSpec((B, 1), lambda i: (0, 0)),
            pl.BlockSpec((B, 1), lambda i: (0, 0)),
            pl.BlockSpec((2 * B, B), lambda i: (0, 0)),
            pl.BlockSpec((BLK, 3), lambda i: (i, 0)),
            pl.BlockSpec((BLK, 3), lambda i: (i, 0)),
            pl.BlockSpec((BLK, 1), lambda i: (i, 0)),
            pl.BlockSpec((BLK, _MAXA), lambda i: (i, 0)),
        ],
        out_specs=[
            pl.BlockSpec((BLK, 3), lambda i: (i, 0)),
            pl.BlockSpec((BLK, 3), lambda i: (i, 0)),
            pl.BlockSpec((BLK, _MAXA), lambda i: (i, 0)),
        ],
        out_shape=[
            jax.ShapeDtypeStruct((N, 3), jnp.float32),
            jax.ShapeDtypeStruct((N, 3), jnp.float32),
            jax.ShapeDtypeStruct((N, _MAXA), jnp.float32),
        ],
        scratch_shapes=[pltpu.VMEM((B, 8), jnp.float32)],
    )(starts_row, ends_row, inv_na_col, alpha_col, sigma_col, sums,
      r, eps, Z.reshape(N, 1), composition_probs)

    return rt, a_s, epsr, t[:, None], s[:, None]
